# Initial kernel scaffold; baseline (speedup 1.0000x reference)
import functools

import jax
import jax.numpy as jnp
from jax import lax
from jax.experimental import pallas as pl
from jax.experimental.pallas import tpu as pltpu
from jax.experimental.pallas import tpu_sc as plsc

N = 100000
D = 32
E = 1600000
EPS = 1e-5

DHALF = N // 2
DUMMY = DHALF
ACC_ROWS = 50048
DEG_ROWS = 51200

LANES = 128
R_TOT = 1601536 // LANES
R_TILE = R_TOT // 16
CHUNK = 17
NCHUNK = R_TILE // CHUNK
E_PAD = R_TOT * LANES

_mesh = plsc.VectorSubcoreMesh(core_axis_name="c", subcore_axis_name="s")


def _zero16():
    return jnp.zeros((16,), jnp.float32)


@functools.partial(
    pl.kernel,
    out_type=[
        jax.ShapeDtypeStruct((2, R_TOT, LANES), jnp.int32),
        jax.ShapeDtypeStruct((2, DEG_ROWS), jnp.float32),
    ],
    mesh=_mesh,
    scratch_types=[
        pltpu.VMEM_SHARED((DEG_ROWS,), jnp.float32),
        pltpu.VMEM((3200,), jnp.float32),
        pltpu.VMEM((CHUNK, LANES), jnp.int32),
        pltpu.VMEM((CHUNK, LANES), jnp.int32),
        pltpu.VMEM((CHUNK, LANES), jnp.int32),
        pltpu.VMEM((LANES,), jnp.float32),
    ],
)
def _sc_precompute(row2d, col2d, colloc_out, deg_out,
                   deg_sp, zdeg, rbuf, cbuf, sbuf, ones_v):
    ci = lax.axis_index("c")
    s = lax.axis_index("s")

    @pl.loop(0, 200)
    def _zl(t):
        off = pl.multiple_of(t * 16, 16)
        zdeg[pl.ds(off, 16)] = _zero16()

    for v in range(LANES // 16):
        ones_v[pl.ds(v * 16, 16)] = jnp.full((16,), 1.0, jnp.float32)

    dbase = pl.multiple_of(s * 3200, 8)
    pltpu.sync_copy(zdeg, deg_sp.at[pl.ds(dbase, 3200)])
    plsc.subcore_barrier()

    @pl.loop(0, NCHUNK)
    def _chunk(k):
        rb = s * R_TILE + k * CHUNK
        pltpu.sync_copy(row2d.at[pl.ds(rb, CHUNK)], rbuf)
        pltpu.sync_copy(col2d.at[pl.ds(rb, CHUNK)], cbuf)

        @pl.loop(0, CHUNK)
        def _row(j):
            @pl.loop(0, LANES // 16)
            def _vec(v):
                off = pl.multiple_of(v * 16, 16)
                rv = rbuf[j, pl.ds(off, 16)]
                cv = cbuf[j, pl.ds(off, 16)]
                loc = cv - ci * DHALF
                valid = (loc >= 0) & (loc < DHALF) & (rv != cv)
                sbuf[j, pl.ds(off, 16)] = jnp.where(valid, loc, DUMMY)

        pltpu.sync_copy(sbuf, colloc_out.at[ci, pl.ds(rb, CHUNK)])
        for j in range(CHUNK):
            pltpu.sync_copy(ones_v, deg_sp.at[sbuf.at[j]], add=True)

    plsc.subcore_barrier()
    pltpu.sync_copy(deg_sp.at[pl.ds(dbase, 3200)],
                    deg_out.at[ci, pl.ds(dbase, 3200)])


@functools.partial(
    pl.kernel,
    out_type=jax.ShapeDtypeStruct((N, D), jnp.float32),
    mesh=_mesh,
    scratch_types=[
        pltpu.VMEM_SHARED((ACC_ROWS, D), jnp.float32),
        pltpu.VMEM((625, D), jnp.float32),
        pltpu.VMEM((CHUNK, LANES), jnp.int32),
        pltpu.VMEM((CHUNK, LANES), jnp.int32),
        pltpu.VMEM((CHUNK * LANES, D), jnp.float32),
        pltpu.SemaphoreType.DMA,
    ],
)
def _sc_agg(x_hbm, row2d, colloc, out_hbm, acc, zbuf, gi, si, rows, sem):
    ci = lax.axis_index("c")
    s = lax.axis_index("s")

    @pl.loop(0, 625)
    def _zl(r):
        zbuf[r, pl.ds(0, 16)] = _zero16()
        zbuf[r, pl.ds(16, 16)] = _zero16()

    for q in range(5):
        pltpu.sync_copy(zbuf, acc.at[pl.ds(s * 3125 + q * 625, 625)])
    plsc.subcore_barrier()

    @pl.loop(0, NCHUNK)
    def _chunk(k):
        rb = s * R_TILE + k * CHUNK
        pltpu.sync_copy(row2d.at[pl.ds(rb, CHUNK)], gi)
        pltpu.sync_copy(colloc.at[ci, pl.ds(rb, CHUNK)], si)
        descs = [
            pltpu.async_copy(x_hbm.at[gi.at[j]],
                             rows.at[pl.ds(j * LANES, LANES)], sem)
            for j in range(CHUNK)
        ]
        for d in descs:
            d.wait()
        for j in range(CHUNK):
            pltpu.sync_copy(rows.at[pl.ds(j * LANES, LANES)],
                            acc.at[si.at[j]], add=True)

    plsc.subcore_barrier()
    pltpu.sync_copy(acc.at[pl.ds(s * 3125, 3125)],
                    out_hbm.at[pl.ds(ci * DHALF + s * 3125, 3125)])


_BN = 10000
_GRID = N // _BN


def _combine_body(act, x_ref, raw_ref, cnt_ref, wo_ref, b_ref, wr_ref,
                  y_ref, st_ref, acc_ref):
    i = pl.program_id(0)
    xb = x_ref[...]
    inv = 1.0 / (cnt_ref[...] + 1.0)
    t = (raw_ref[...] + xb) * inv
    h = (jnp.dot(t, wo_ref[...], preferred_element_type=jnp.float32)
         + b_ref[...]
         + jnp.dot(xb, wr_ref[...], preferred_element_type=jnp.float32))
    if act:
        h = jnp.where(h >= 0, h, 0.01 * h)
    y = h + xb
    y_ref[...] = y

    @pl.when(i == 0)
    def _init():
        acc_ref[...] = jnp.zeros_like(acc_ref)

    acc_ref[0:1, :] = acc_ref[0:1, :] + jnp.sum(y, axis=0, keepdims=True)
    acc_ref[1:2, :] = acc_ref[1:2, :] + jnp.sum(y * y, axis=0, keepdims=True)

    @pl.when(i == _GRID - 1)
    def _fin():
        st_ref[...] = acc_ref[...]


def _tc_combine(x, raw, cnt, wo, b, wr, act):
    return pl.pallas_call(
        functools.partial(_combine_body, act),
        grid=(_GRID,),
        in_specs=[
            pl.BlockSpec((_BN, D), lambda i: (i, 0)),
            pl.BlockSpec((_BN, D), lambda i: (i, 0)),
            pl.BlockSpec((_BN, 1), lambda i: (i, 0)),
            pl.BlockSpec((D, D), lambda i: (0, 0)),
            pl.BlockSpec((1, D), lambda i: (0, 0)),
            pl.BlockSpec((D, D), lambda i: (0, 0)),
        ],
        out_specs=[
            pl.BlockSpec((_BN, D), lambda i: (i, 0)),
            pl.BlockSpec((2, D), lambda i: (0, 0)),
        ],
        out_shape=[
            jax.ShapeDtypeStruct((N, D), jnp.float32),
            jax.ShapeDtypeStruct((2, D), jnp.float32),
        ],
        scratch_shapes=[pltpu.VMEM((2, D), jnp.float32)],
    )(x, raw, cnt, wo, b, wr)


def _norm_body(y_ref, st_ref, g_ref, be_ref, o_ref):
    st = st_ref[...]
    mean = st[0:1, :] / N
    var = st[1:2, :] / N - mean * mean
    scale = g_ref[...] / jnp.sqrt(var + EPS)
    o_ref[...] = (y_ref[...] - mean) * scale + be_ref[...]


def _tc_norm(y, st, g, be):
    return pl.pallas_call(
        _norm_body,
        grid=(_GRID,),
        in_specs=[
            pl.BlockSpec((_BN, D), lambda i: (i, 0)),
            pl.BlockSpec((2, D), lambda i: (0, 0)),
            pl.BlockSpec((1, D), lambda i: (0, 0)),
            pl.BlockSpec((1, D), lambda i: (0, 0)),
        ],
        out_specs=pl.BlockSpec((_BN, D), lambda i: (i, 0)),
        out_shape=jax.ShapeDtypeStruct((N, D), jnp.float32),
    )(y, st, g, be)


def kernel(patch_embs, edge_index, edge_attr,
           W1_out, b1_out, W1_root, g1, be1,
           W2_out, b2_out, W2_root, g2, be2,
           W3_out, b3_out, W3_root, g3, be3):
    row = edge_index[0].astype(jnp.int32)
    col = edge_index[1].astype(jnp.int32)
    pad = E_PAD - E
    row_p = jnp.concatenate([row, jnp.zeros((pad,), jnp.int32)])
    col_p = jnp.concatenate([col, jnp.full((pad,), N, jnp.int32)])
    row2d = row_p.reshape(R_TOT, LANES)
    col2d = col_p.reshape(R_TOT, LANES)

    colloc, degp = _sc_precompute(row2d, col2d)
    cnt = degp[:, :DHALF].reshape(N, 1)

    x = patch_embs
    layers = [
        (W1_out, b1_out, W1_root, g1, be1, True),
        (W2_out, b2_out, W2_root, g2, be2, True),
        (W3_out, b3_out, W3_root, g3, be3, False),
    ]
    for wo, b, wr, g, be, act in layers:
        raw = _sc_agg(x, row2d, colloc)
        y, st = _tc_combine(x, raw, cnt, wo, b.reshape(1, D), wr, act)
        x = _tc_norm(y, st, g.reshape(1, D), be.reshape(1, D))
    return x


# speedup vs baseline: 7.5767x; 7.5767x over previous
import functools

import jax
import jax.numpy as jnp
from jax import lax
from jax.experimental import pallas as pl
from jax.experimental.pallas import tpu as pltpu
from jax.experimental.pallas import tpu_sc as plsc

N = 100000
D = 32
E = 1600000
EPS = 1e-5

DHALF = N // 2
DUMMY = DHALF
ACC_ROWS = 50048
DEG_ROWS = 51200

LANES = 128
R_TILE = 784
R_TOT = R_TILE * 16
CHUNK = 4
NCHUNK = R_TILE // CHUNK
E_PAD = R_TOT * LANES
TROWS = ACC_ROWS // 16

_mesh = plsc.VectorSubcoreMesh(core_axis_name="c", subcore_axis_name="s")


def _zero16():
    return jnp.zeros((16,), jnp.float32)


@functools.partial(
    pl.kernel,
    out_type=[
        jax.ShapeDtypeStruct((2, R_TOT, LANES), jnp.int32),
        jax.ShapeDtypeStruct((2, DEG_ROWS), jnp.float32),
    ],
    mesh=_mesh,
    compiler_params=pltpu.CompilerParams(use_tc_tiling_on_sc=False),
    scratch_types=[
        pltpu.VMEM_SHARED((DEG_ROWS,), jnp.float32),
        pltpu.VMEM((3200,), jnp.float32),
        pltpu.VMEM((CHUNK, LANES), jnp.int32),
        pltpu.VMEM((CHUNK, LANES), jnp.int32),
        pltpu.VMEM((CHUNK, LANES), jnp.int32),
        pltpu.VMEM((LANES,), jnp.float32),
    ],
)
def _sc_precompute(row2d, col2d, colloc_out, deg_out,
                   deg_sp, zdeg, rbuf, cbuf, sbuf, ones_v):
    ci = lax.axis_index("c")
    s = lax.axis_index("s")

    @pl.loop(0, 200)
    def _zl(t):
        off = pl.multiple_of(t * 16, 16)
        zdeg[pl.ds(off, 16)] = _zero16()

    for v in range(LANES // 16):
        ones_v[pl.ds(v * 16, 16)] = jnp.full((16,), 1.0, jnp.float32)

    dbase = pl.multiple_of(s * 3200, 8)
    pltpu.sync_copy(zdeg, deg_sp.at[pl.ds(dbase, 3200)])
    plsc.subcore_barrier()

    @pl.loop(0, NCHUNK)
    def _chunk(k):
        rb = s * R_TILE + k * CHUNK
        pltpu.sync_copy(row2d.at[pl.ds(rb, CHUNK)], rbuf)
        pltpu.sync_copy(col2d.at[pl.ds(rb, CHUNK)], cbuf)

        @pl.loop(0, CHUNK)
        def _row(j):
            @pl.loop(0, LANES // 16)
            def _vec(v):
                off = pl.multiple_of(v * 16, 16)
                rv = rbuf[j, pl.ds(off, 16)]
                cv = cbuf[j, pl.ds(off, 16)]
                loc = cv - ci * DHALF
                valid = (loc >= 0) & (loc < DHALF) & (rv != cv)
                sbuf[j, pl.ds(off, 16)] = jnp.where(valid, loc, DUMMY)

        pltpu.sync_copy(sbuf, colloc_out.at[ci, pl.ds(rb, CHUNK)])
        for j in range(CHUNK):
            pltpu.sync_copy(ones_v, deg_sp.at[sbuf.at[j]], add=True)

    plsc.subcore_barrier()
    pltpu.sync_copy(deg_sp.at[pl.ds(dbase, 3200)],
                    deg_out.at[ci, pl.ds(dbase, 3200)])


@functools.partial(
    pl.kernel,
    out_type=jax.ShapeDtypeStruct((2 * ACC_ROWS, D), jnp.float32),
    mesh=_mesh,
    compiler_params=pltpu.CompilerParams(use_tc_tiling_on_sc=False),
    scratch_types=[
        pltpu.VMEM_SHARED((ACC_ROWS, D), jnp.float32),
        pltpu.VMEM((184, D), jnp.float32),
        pltpu.VMEM((CHUNK, LANES), jnp.int32),
        pltpu.VMEM((CHUNK, LANES), jnp.int32),
        pltpu.VMEM((CHUNK * LANES, D), jnp.float32),
        pltpu.SemaphoreType.DMA,
    ],
)
def _sc_agg(x_hbm, row2d, colloc, out_hbm, acc, zbuf, gi, si, rows, sem):
    ci = lax.axis_index("c")
    s = lax.axis_index("s")

    @pl.loop(0, 184)
    def _zl(r):
        zbuf[r, pl.ds(0, 16)] = _zero16()
        zbuf[r, pl.ds(16, 16)] = _zero16()

    for q in range(17):
        pltpu.sync_copy(zbuf, acc.at[pl.ds(s * TROWS + q * 184, 184)])
    plsc.subcore_barrier()

    @pl.loop(0, NCHUNK)
    def _chunk(k):
        rb = s * R_TILE + k * CHUNK
        pltpu.sync_copy(row2d.at[pl.ds(rb, CHUNK)], gi)
        pltpu.sync_copy(colloc.at[ci, pl.ds(rb, CHUNK)], si)
        descs = [
            pltpu.async_copy(x_hbm.at[gi.at[j]],
                             rows.at[pl.ds(j * LANES, LANES)], sem)
            for j in range(CHUNK)
        ]
        for d in descs:
            d.wait()
        for j in range(CHUNK):
            pltpu.sync_copy(rows.at[pl.ds(j * LANES, LANES)],
                            acc.at[si.at[j]], add=True)

    plsc.subcore_barrier()
    pltpu.sync_copy(acc.at[pl.ds(s * TROWS, TROWS)],
                    out_hbm.at[pl.ds(ci * ACC_ROWS + s * TROWS, TROWS)])


_BN = 10000
_GRID = N // _BN


def _combine_body(act, x_ref, raw_ref, cnt_ref, wo_ref, b_ref, wr_ref,
                  y_ref, st_ref, acc_ref):
    i = pl.program_id(0)
    xb = x_ref[...]
    inv = 1.0 / (cnt_ref[...] + 1.0)
    t = (raw_ref[...] + xb) * inv
    h = (jnp.dot(t, wo_ref[...], preferred_element_type=jnp.float32)
         + b_ref[...]
         + jnp.dot(xb, wr_ref[...], preferred_element_type=jnp.float32))
    if act:
        h = jnp.where(h >= 0, h, 0.01 * h)
    y = h + xb
    y_ref[...] = y

    @pl.when(i == 0)
    def _init():
        acc_ref[...] = jnp.zeros_like(acc_ref)

    acc_ref[0:1, :] = acc_ref[0:1, :] + jnp.sum(y, axis=0, keepdims=True)
    acc_ref[1:2, :] = acc_ref[1:2, :] + jnp.sum(y * y, axis=0, keepdims=True)

    @pl.when(i == _GRID - 1)
    def _fin():
        st_ref[...] = acc_ref[...]


def _tc_combine(x, raw, cnt, wo, b, wr, act):
    return pl.pallas_call(
        functools.partial(_combine_body, act),
        grid=(_GRID,),
        in_specs=[
            pl.BlockSpec((_BN, D), lambda i: (i, 0)),
            pl.BlockSpec((_BN, D), lambda i: (i, 0)),
            pl.BlockSpec((_BN, 1), lambda i: (i, 0)),
            pl.BlockSpec((D, D), lambda i: (0, 0)),
            pl.BlockSpec((1, D), lambda i: (0, 0)),
            pl.BlockSpec((D, D), lambda i: (0, 0)),
        ],
        out_specs=[
            pl.BlockSpec((_BN, D), lambda i: (i, 0)),
            pl.BlockSpec((2, D), lambda i: (0, 0)),
        ],
        out_shape=[
            jax.ShapeDtypeStruct((N, D), jnp.float32),
            jax.ShapeDtypeStruct((2, D), jnp.float32),
        ],
        scratch_shapes=[pltpu.VMEM((2, D), jnp.float32)],
    )(x, raw, cnt, wo, b, wr)


def _norm_body(y_ref, st_ref, g_ref, be_ref, o_ref):
    st = st_ref[...]
    mean = st[0:1, :] / N
    var = st[1:2, :] / N - mean * mean
    scale = g_ref[...] / jnp.sqrt(var + EPS)
    o_ref[...] = (y_ref[...] - mean) * scale + be_ref[...]


def _tc_norm(y, st, g, be):
    return pl.pallas_call(
        _norm_body,
        grid=(_GRID,),
        in_specs=[
            pl.BlockSpec((_BN, D), lambda i: (i, 0)),
            pl.BlockSpec((2, D), lambda i: (0, 0)),
            pl.BlockSpec((1, D), lambda i: (0, 0)),
            pl.BlockSpec((1, D), lambda i: (0, 0)),
        ],
        out_specs=pl.BlockSpec((_BN, D), lambda i: (i, 0)),
        out_shape=jax.ShapeDtypeStruct((N, D), jnp.float32),
    )(y, st, g, be)


def kernel(patch_embs, edge_index, edge_attr,
           W1_out, b1_out, W1_root, g1, be1,
           W2_out, b2_out, W2_root, g2, be2,
           W3_out, b3_out, W3_root, g3, be3):
    row = edge_index[0].astype(jnp.int32)
    col = edge_index[1].astype(jnp.int32)
    pad = E_PAD - E
    row_p = jnp.concatenate([row, jnp.zeros((pad,), jnp.int32)])
    col_p = jnp.concatenate([col, jnp.full((pad,), N, jnp.int32)])
    row2d = row_p.reshape(R_TOT, LANES)
    col2d = col_p.reshape(R_TOT, LANES)

    colloc, degp = _sc_precompute(row2d, col2d)
    cnt = degp[:, :DHALF].reshape(N, 1)

    x = patch_embs
    layers = [
        (W1_out, b1_out, W1_root, g1, be1, True),
        (W2_out, b2_out, W2_root, g2, be2, True),
        (W3_out, b3_out, W3_root, g3, be3, False),
    ]
    for wo, b, wr, g, be, act in layers:
        raw_pad = _sc_agg(x, row2d, colloc)
        raw = jnp.concatenate([raw_pad[:DHALF],
                               raw_pad[ACC_ROWS:ACC_ROWS + DHALF]], axis=0)
        y, st = _tc_combine(x, raw, cnt, wo, b.reshape(1, D), wr, act)
        x = _tc_norm(y, st, g.reshape(1, D), be.reshape(1, D))
    return x


# speedup vs baseline: 11.3533x; 1.4984x over previous
import functools

import jax
import jax.numpy as jnp
from jax import lax
from jax.experimental import pallas as pl
from jax.experimental.pallas import tpu as pltpu
from jax.experimental.pallas import tpu_sc as plsc

N = 100000
D = 32
DH = D // 2
E = 1600000
EPS = 1e-5

DUMMY = N
ACC_ROWS = 100096
TROWS = ACC_ROWS // 16

DHALF = N // 2
DEG_DUMMY = DHALF
DEG_ROWS = 51200

LANES = 128
R_TILE = 784
R_TOT = R_TILE * 16
CHUNK = 8
NCHUNK = R_TILE // CHUNK
E_PAD = R_TOT * LANES

_mesh = plsc.VectorSubcoreMesh(core_axis_name="c", subcore_axis_name="s")


def _zero16():
    return jnp.zeros((16,), jnp.float32)


@functools.partial(
    pl.kernel,
    out_type=[
        jax.ShapeDtypeStruct((R_TOT, LANES), jnp.int32),
        jax.ShapeDtypeStruct((2, DEG_ROWS), jnp.float32),
    ],
    mesh=_mesh,
    compiler_params=pltpu.CompilerParams(use_tc_tiling_on_sc=False),
    scratch_types=[
        pltpu.VMEM_SHARED((DEG_ROWS,), jnp.float32),
        pltpu.VMEM((3200,), jnp.float32),
        pltpu.VMEM((CHUNK, LANES), jnp.int32),
        pltpu.VMEM((CHUNK, LANES), jnp.int32),
        pltpu.VMEM((CHUNK, LANES), jnp.int32),
        pltpu.VMEM((CHUNK, LANES), jnp.int32),
        pltpu.VMEM((LANES,), jnp.float32),
    ],
)
def _sc_precompute(row2d, col2d, colloc_out, deg_out,
                   deg_sp, zdeg, rbuf, cbuf, sbuf, dbuf, ones_v):
    ci = lax.axis_index("c")
    s = lax.axis_index("s")

    @pl.loop(0, 200)
    def _zl(t):
        off = pl.multiple_of(t * 16, 16)
        zdeg[pl.ds(off, 16)] = _zero16()

    for v in range(LANES // 16):
        ones_v[pl.ds(v * 16, 16)] = jnp.full((16,), 1.0, jnp.float32)

    dbase = pl.multiple_of(s * 3200, 8)
    pltpu.sync_copy(zdeg, deg_sp.at[pl.ds(dbase, 3200)])
    plsc.subcore_barrier()

    @pl.loop(0, NCHUNK)
    def _chunk(k):
        rb = s * R_TILE + k * CHUNK
        pltpu.sync_copy(row2d.at[pl.ds(rb, CHUNK)], rbuf)
        pltpu.sync_copy(col2d.at[pl.ds(rb, CHUNK)], cbuf)

        @pl.loop(0, CHUNK)
        def _row(j):
            @pl.loop(0, LANES // 16)
            def _vec(v):
                off = pl.multiple_of(v * 16, 16)
                rv = rbuf[j, pl.ds(off, 16)]
                cv = cbuf[j, pl.ds(off, 16)]
                ok = rv != cv
                sbuf[j, pl.ds(off, 16)] = jnp.where(ok, cv, DUMMY)
                loc = cv - ci * DHALF
                okl = (loc >= 0) & (loc < DHALF) & ok
                dbuf[j, pl.ds(off, 16)] = jnp.where(okl, loc, DEG_DUMMY)

        @pl.when(ci == 0)
        def _store():
            pltpu.sync_copy(sbuf, colloc_out.at[pl.ds(rb, CHUNK)])

        for j in range(CHUNK):
            pltpu.sync_copy(ones_v, deg_sp.at[dbuf.at[j]], add=True)

    plsc.subcore_barrier()
    pltpu.sync_copy(deg_sp.at[pl.ds(dbase, 3200)],
                    deg_out.at[ci, pl.ds(dbase, 3200)])


@functools.partial(
    pl.kernel,
    out_type=jax.ShapeDtypeStruct((2, ACC_ROWS, DH), jnp.float32),
    mesh=_mesh,
    compiler_params=pltpu.CompilerParams(use_tc_tiling_on_sc=False),
    scratch_types=[
        pltpu.VMEM_SHARED((ACC_ROWS, DH), jnp.float32),
        pltpu.VMEM((184, DH), jnp.float32),
        pltpu.VMEM((CHUNK, LANES), jnp.int32),
        pltpu.VMEM((CHUNK, LANES), jnp.int32),
        pltpu.VMEM((CHUNK * LANES, DH), jnp.float32),
        pltpu.SemaphoreType.DMA,
    ],
)
def _sc_agg(x3_hbm, row2d, colloc, out_hbm, acc, zbuf, gi, si, rows, sem):
    ci = lax.axis_index("c")
    s = lax.axis_index("s")
    xv = x3_hbm.at[ci]

    @pl.loop(0, 184)
    def _zl(r):
        zbuf[r, pl.ds(0, 16)] = _zero16()

    for q in range(TROWS // 184):
        pltpu.sync_copy(zbuf, acc.at[pl.ds(s * TROWS + q * 184, 184)])
    plsc.subcore_barrier()

    @pl.loop(0, NCHUNK)
    def _chunk(k):
        rb = s * R_TILE + k * CHUNK
        pltpu.sync_copy(row2d.at[pl.ds(rb, CHUNK)], gi)
        pltpu.sync_copy(colloc.at[pl.ds(rb, CHUNK)], si)
        descs = [
            pltpu.async_copy(xv.at[gi.at[j]],
                             rows.at[pl.ds(j * LANES, LANES)], sem)
            for j in range(CHUNK)
        ]
        for d in descs:
            d.wait()
        for j in range(CHUNK):
            pltpu.sync_copy(rows.at[pl.ds(j * LANES, LANES)],
                            acc.at[si.at[j]], add=True)

    plsc.subcore_barrier()
    pltpu.sync_copy(acc.at[pl.ds(s * TROWS, TROWS)],
                    out_hbm.at[ci, pl.ds(s * TROWS, TROWS)])


_BN = 2000
_GRID = N // _BN


def _combine_body(act, x3_ref, r3_ref, cnt_ref, wo_ref, b_ref, wr_ref,
                  y_ref, st_ref, acc_ref):
    i = pl.program_id(0)
    xb = jnp.concatenate([x3_ref[0], x3_ref[1]], axis=1)
    rawb = jnp.concatenate([r3_ref[0], r3_ref[1]], axis=1)
    inv = 1.0 / (cnt_ref[...] + 1.0)
    t = (rawb + xb) * inv
    h = (jnp.dot(t, wo_ref[...], preferred_element_type=jnp.float32)
         + b_ref[...]
         + jnp.dot(xb, wr_ref[...], preferred_element_type=jnp.float32))
    if act:
        h = jnp.where(h >= 0, h, 0.01 * h)
    y = h + xb
    y_ref[...] = y

    @pl.when(i == 0)
    def _init():
        acc_ref[...] = jnp.zeros_like(acc_ref)

    acc_ref[0:1, :] = acc_ref[0:1, :] + jnp.sum(y, axis=0, keepdims=True)
    acc_ref[1:2, :] = acc_ref[1:2, :] + jnp.sum(y * y, axis=0, keepdims=True)

    @pl.when(i == _GRID - 1)
    def _fin():
        st_ref[...] = acc_ref[...]


def _tc_combine(x3, raw3, cnt, wo, b, wr, act):
    return pl.pallas_call(
        functools.partial(_combine_body, act),
        grid=(_GRID,),
        in_specs=[
            pl.BlockSpec((2, _BN, DH), lambda i: (0, i, 0)),
            pl.BlockSpec((2, _BN, DH), lambda i: (0, i, 0)),
            pl.BlockSpec((_BN, 1), lambda i: (i, 0)),
            pl.BlockSpec((D, D), lambda i: (0, 0)),
            pl.BlockSpec((1, D), lambda i: (0, 0)),
            pl.BlockSpec((D, D), lambda i: (0, 0)),
        ],
        out_specs=[
            pl.BlockSpec((_BN, D), lambda i: (i, 0)),
            pl.BlockSpec((2, D), lambda i: (0, 0)),
        ],
        out_shape=[
            jax.ShapeDtypeStruct((N, D), jnp.float32),
            jax.ShapeDtypeStruct((2, D), jnp.float32),
        ],
        scratch_shapes=[pltpu.VMEM((2, D), jnp.float32)],
    )(x3, raw3, cnt, wo, b, wr)


def _norm_body_split(y_ref, st_ref, g_ref, be_ref, o3_ref):
    st = st_ref[...]
    mean = st[0:1, :] / N
    var = st[1:2, :] / N - mean * mean
    scale = g_ref[...] / jnp.sqrt(var + EPS)
    z = (y_ref[...] - mean) * scale + be_ref[...]
    o3_ref[0] = z[:, :DH]
    o3_ref[1] = z[:, DH:]


def _norm_body_dense(y_ref, st_ref, g_ref, be_ref, o_ref):
    st = st_ref[...]
    mean = st[0:1, :] / N
    var = st[1:2, :] / N - mean * mean
    scale = g_ref[...] / jnp.sqrt(var + EPS)
    o_ref[...] = (y_ref[...] - mean) * scale + be_ref[...]


def _tc_norm(y, st, g, be, split):
    if split:
        body = _norm_body_split
        out_spec = pl.BlockSpec((2, _BN, DH), lambda i: (0, i, 0))
        out_shape = jax.ShapeDtypeStruct((2, N, DH), jnp.float32)
    else:
        body = _norm_body_dense
        out_spec = pl.BlockSpec((_BN, D), lambda i: (i, 0))
        out_shape = jax.ShapeDtypeStruct((N, D), jnp.float32)
    return pl.pallas_call(
        body,
        grid=(_GRID,),
        in_specs=[
            pl.BlockSpec((_BN, D), lambda i: (i, 0)),
            pl.BlockSpec((2, D), lambda i: (0, 0)),
            pl.BlockSpec((1, D), lambda i: (0, 0)),
            pl.BlockSpec((1, D), lambda i: (0, 0)),
        ],
        out_specs=out_spec,
        out_shape=out_shape,
    )(y, st, g, be)


def kernel(patch_embs, edge_index, edge_attr,
           W1_out, b1_out, W1_root, g1, be1,
           W2_out, b2_out, W2_root, g2, be2,
           W3_out, b3_out, W3_root, g3, be3):
    row = edge_index[0].astype(jnp.int32)
    col = edge_index[1].astype(jnp.int32)
    pad = E_PAD - E
    row_p = jnp.concatenate([row, jnp.zeros((pad,), jnp.int32)])
    col_p = jnp.concatenate([col, jnp.full((pad,), N, jnp.int32)])
    row2d = row_p.reshape(R_TOT, LANES)
    col2d = col_p.reshape(R_TOT, LANES)

    colloc, degp = _sc_precompute(row2d, col2d)
    cnt = degp[:, :DHALF].reshape(N, 1)

    x3 = jnp.stack([patch_embs[:, :DH], patch_embs[:, DH:]])
    layers = [
        (W1_out, b1_out, W1_root, g1, be1, True),
        (W2_out, b2_out, W2_root, g2, be2, True),
        (W3_out, b3_out, W3_root, g3, be3, False),
    ]
    for li, (wo, b, wr, g, be, act) in enumerate(layers):
        raw3 = _sc_agg(x3, row2d, colloc)
        y, st = _tc_combine(x3, raw3, cnt, wo, b.reshape(1, D), wr, act)
        x3 = _tc_norm(y, st, g.reshape(1, D), be.reshape(1, D),
                      split=(li < 2))
    return x3


# speedup vs baseline: 12.3269x; 1.0858x over previous
import functools

import jax
import jax.numpy as jnp
from jax import lax
from jax.experimental import pallas as pl
from jax.experimental.pallas import tpu as pltpu
from jax.experimental.pallas import tpu_sc as plsc

N = 100000
D = 32
DH = D // 2
E = 1600000
EPS = 1e-5

DUMMY = N
ACC_ROWS = 100096
TROWS = ACC_ROWS // 16

DHALF = N // 2
DEG_DUMMY = DHALF
DEG_ROWS = 51200

LANES = 128
R_TILE = 784
R_TOT = R_TILE * 16
CHUNK = 4
NCHUNK = R_TILE // CHUNK
NPAIR = NCHUNK // 2
E_PAD = R_TOT * LANES

_mesh = plsc.VectorSubcoreMesh(core_axis_name="c", subcore_axis_name="s")


def _zero16():
    return jnp.zeros((16,), jnp.float32)


@functools.partial(
    pl.kernel,
    out_type=[
        jax.ShapeDtypeStruct((R_TOT, LANES), jnp.int32),
        jax.ShapeDtypeStruct((2, DEG_ROWS), jnp.float32),
    ],
    mesh=_mesh,
    compiler_params=pltpu.CompilerParams(use_tc_tiling_on_sc=False),
    scratch_types=[
        pltpu.VMEM_SHARED((DEG_ROWS,), jnp.float32),
        pltpu.VMEM((3200,), jnp.float32),
        pltpu.VMEM((CHUNK, LANES), jnp.int32),
        pltpu.VMEM((CHUNK, LANES), jnp.int32),
        pltpu.VMEM((CHUNK, LANES), jnp.int32),
        pltpu.VMEM((CHUNK, LANES), jnp.int32),
        pltpu.VMEM((LANES,), jnp.float32),
    ],
)
def _sc_precompute(row2d, col2d, colloc_out, deg_out,
                   deg_sp, zdeg, rbuf, cbuf, sbuf, dbuf, ones_v):
    ci = lax.axis_index("c")
    s = lax.axis_index("s")

    @pl.loop(0, 200)
    def _zl(t):
        off = pl.multiple_of(t * 16, 16)
        zdeg[pl.ds(off, 16)] = _zero16()

    for v in range(LANES // 16):
        ones_v[pl.ds(v * 16, 16)] = jnp.full((16,), 1.0, jnp.float32)

    dbase = pl.multiple_of(s * 3200, 8)
    pltpu.sync_copy(zdeg, deg_sp.at[pl.ds(dbase, 3200)])
    plsc.subcore_barrier()

    @pl.loop(0, NCHUNK)
    def _chunk(k):
        rb = s * R_TILE + k * CHUNK
        pltpu.sync_copy(row2d.at[pl.ds(rb, CHUNK)], rbuf)
        pltpu.sync_copy(col2d.at[pl.ds(rb, CHUNK)], cbuf)

        @pl.loop(0, CHUNK)
        def _row(j):
            @pl.loop(0, LANES // 16)
            def _vec(v):
                off = pl.multiple_of(v * 16, 16)
                rv = rbuf[j, pl.ds(off, 16)]
                cv = cbuf[j, pl.ds(off, 16)]
                ok = rv != cv
                sbuf[j, pl.ds(off, 16)] = jnp.where(ok, cv, DUMMY)
                loc = cv - ci * DHALF
                okl = (loc >= 0) & (loc < DHALF) & ok
                dbuf[j, pl.ds(off, 16)] = jnp.where(okl, loc, DEG_DUMMY)

        @pl.when(ci == 0)
        def _store():
            pltpu.sync_copy(sbuf, colloc_out.at[pl.ds(rb, CHUNK)])

        for j in range(CHUNK):
            pltpu.sync_copy(ones_v, deg_sp.at[dbuf.at[j]], add=True)

    plsc.subcore_barrier()
    pltpu.sync_copy(deg_sp.at[pl.ds(dbase, 3200)],
                    deg_out.at[ci, pl.ds(dbase, 3200)])


@functools.partial(
    pl.kernel,
    out_type=jax.ShapeDtypeStruct((2, ACC_ROWS, DH), jnp.float32),
    mesh=_mesh,
    compiler_params=pltpu.CompilerParams(use_tc_tiling_on_sc=False),
    scratch_types=[
        pltpu.VMEM_SHARED((ACC_ROWS, DH), jnp.float32),
        pltpu.VMEM((184, DH), jnp.float32),
        pltpu.VMEM((2 * CHUNK, LANES), jnp.int32),
        pltpu.VMEM((2 * CHUNK, LANES), jnp.int32),
        pltpu.VMEM((2 * CHUNK * LANES, DH), jnp.float32),
        pltpu.SemaphoreType.DMA,
        pltpu.SemaphoreType.DMA,
    ],
)
def _sc_agg(x3_hbm, row2d, colloc, out_hbm, acc, zbuf, gi, si, rows,
            sem0, sem1):
    ci = lax.axis_index("c")
    s = lax.axis_index("s")
    xv = x3_hbm.at[ci]
    sems = (sem0, sem1)

    @pl.loop(0, 184)
    def _zl(r):
        zbuf[r, pl.ds(0, 16)] = _zero16()

    for q in range(TROWS // 184):
        pltpu.sync_copy(zbuf, acc.at[pl.ds(s * TROWS + q * 184, 184)])
    plsc.subcore_barrier()

    def _fire(b, k):
        rb = s * R_TILE + k * CHUNK
        pltpu.sync_copy(row2d.at[pl.ds(rb, CHUNK)],
                        gi.at[pl.ds(b * CHUNK, CHUNK)])
        pltpu.sync_copy(colloc.at[pl.ds(rb, CHUNK)],
                        si.at[pl.ds(b * CHUNK, CHUNK)])
        for j in range(CHUNK):
            pltpu.async_copy(
                xv.at[gi.at[b * CHUNK + j]],
                rows.at[pl.ds((b * CHUNK + j) * LANES, LANES)], sems[b])

    def _drain_and_scatter(b):
        for j in range(CHUNK):
            pltpu.make_async_copy(
                xv.at[gi.at[b * CHUNK + j]],
                rows.at[pl.ds((b * CHUNK + j) * LANES, LANES)],
                sems[b]).wait()
        for j in range(CHUNK):
            pltpu.sync_copy(rows.at[pl.ds((b * CHUNK + j) * LANES, LANES)],
                            acc.at[si.at[b * CHUNK + j]], add=True)

    _fire(0, 0)

    @pl.loop(0, NPAIR)
    def _pair(kk):
        _fire(1, 2 * kk + 1)
        _drain_and_scatter(0)

        @pl.when(kk < NPAIR - 1)
        def _next():
            _fire(0, 2 * kk + 2)

        _drain_and_scatter(1)

    plsc.subcore_barrier()
    pltpu.sync_copy(acc.at[pl.ds(s * TROWS, TROWS)],
                    out_hbm.at[ci, pl.ds(s * TROWS, TROWS)])


_BN = 2000
_GRID = N // _BN


def _combine_body(act, x3_ref, r3_ref, cnt_ref, wo_ref, b_ref, wr_ref,
                  y_ref, st_ref, acc_ref):
    i = pl.program_id(0)
    xb = jnp.concatenate([x3_ref[0], x3_ref[1]], axis=1)
    rawb = jnp.concatenate([r3_ref[0], r3_ref[1]], axis=1)
    inv = 1.0 / (cnt_ref[...] + 1.0)
    t = (rawb + xb) * inv
    h = (jnp.dot(t, wo_ref[...], preferred_element_type=jnp.float32)
         + b_ref[...]
         + jnp.dot(xb, wr_ref[...], preferred_element_type=jnp.float32))
    if act:
        h = jnp.where(h >= 0, h, 0.01 * h)
    y = h + xb
    y_ref[...] = y

    @pl.when(i == 0)
    def _init():
        acc_ref[...] = jnp.zeros_like(acc_ref)

    acc_ref[0:1, :] = acc_ref[0:1, :] + jnp.sum(y, axis=0, keepdims=True)
    acc_ref[1:2, :] = acc_ref[1:2, :] + jnp.sum(y * y, axis=0, keepdims=True)

    @pl.when(i == _GRID - 1)
    def _fin():
        st_ref[...] = acc_ref[...]


def _tc_combine(x3, raw3, cnt, wo, b, wr, act):
    return pl.pallas_call(
        functools.partial(_combine_body, act),
        grid=(_GRID,),
        in_specs=[
            pl.BlockSpec((2, _BN, DH), lambda i: (0, i, 0)),
            pl.BlockSpec((2, _BN, DH), lambda i: (0, i, 0)),
            pl.BlockSpec((_BN, 1), lambda i: (i, 0)),
            pl.BlockSpec((D, D), lambda i: (0, 0)),
            pl.BlockSpec((1, D), lambda i: (0, 0)),
            pl.BlockSpec((D, D), lambda i: (0, 0)),
        ],
        out_specs=[
            pl.BlockSpec((_BN, D), lambda i: (i, 0)),
            pl.BlockSpec((2, D), lambda i: (0, 0)),
        ],
        out_shape=[
            jax.ShapeDtypeStruct((N, D), jnp.float32),
            jax.ShapeDtypeStruct((2, D), jnp.float32),
        ],
        scratch_shapes=[pltpu.VMEM((2, D), jnp.float32)],
    )(x3, raw3, cnt, wo, b, wr)


def _norm_body_split(y_ref, st_ref, g_ref, be_ref, o3_ref):
    st = st_ref[...]
    mean = st[0:1, :] / N
    var = st[1:2, :] / N - mean * mean
    scale = g_ref[...] / jnp.sqrt(var + EPS)
    z = (y_ref[...] - mean) * scale + be_ref[...]
    o3_ref[0] = z[:, :DH]
    o3_ref[1] = z[:, DH:]


def _norm_body_dense(y_ref, st_ref, g_ref, be_ref, o_ref):
    st = st_ref[...]
    mean = st[0:1, :] / N
    var = st[1:2, :] / N - mean * mean
    scale = g_ref[...] / jnp.sqrt(var + EPS)
    o_ref[...] = (y_ref[...] - mean) * scale + be_ref[...]


def _tc_norm(y, st, g, be, split):
    if split:
        body = _norm_body_split
        out_spec = pl.BlockSpec((2, _BN, DH), lambda i: (0, i, 0))
        out_shape = jax.ShapeDtypeStruct((2, N, DH), jnp.float32)
    else:
        body = _norm_body_dense
        out_spec = pl.BlockSpec((_BN, D), lambda i: (i, 0))
        out_shape = jax.ShapeDtypeStruct((N, D), jnp.float32)
    return pl.pallas_call(
        body,
        grid=(_GRID,),
        in_specs=[
            pl.BlockSpec((_BN, D), lambda i: (i, 0)),
            pl.BlockSpec((2, D), lambda i: (0, 0)),
            pl.BlockSpec((1, D), lambda i: (0, 0)),
            pl.BlockSpec((1, D), lambda i: (0, 0)),
        ],
        out_specs=out_spec,
        out_shape=out_shape,
    )(y, st, g, be)


def kernel(patch_embs, edge_index, edge_attr,
           W1_out, b1_out, W1_root, g1, be1,
           W2_out, b2_out, W2_root, g2, be2,
           W3_out, b3_out, W3_root, g3, be3):
    row = edge_index[0].astype(jnp.int32)
    col = edge_index[1].astype(jnp.int32)
    pad = E_PAD - E
    row_p = jnp.concatenate([row, jnp.zeros((pad,), jnp.int32)])
    col_p = jnp.concatenate([col, jnp.full((pad,), N, jnp.int32)])
    row2d = row_p.reshape(R_TOT, LANES)
    col2d = col_p.reshape(R_TOT, LANES)

    colloc, degp = _sc_precompute(row2d, col2d)
    cnt = degp[:, :DHALF].reshape(N, 1)

    x3 = jnp.stack([patch_embs[:, :DH], patch_embs[:, DH:]])
    layers = [
        (W1_out, b1_out, W1_root, g1, be1, True),
        (W2_out, b2_out, W2_root, g2, be2, True),
        (W3_out, b3_out, W3_root, g3, be3, False),
    ]
    for li, (wo, b, wr, g, be, act) in enumerate(layers):
        raw3 = _sc_agg(x3, row2d, colloc)
        y, st = _tc_combine(x3, raw3, cnt, wo, b.reshape(1, D), wr, act)
        x3 = _tc_norm(y, st, g.reshape(1, D), be.reshape(1, D),
                      split=(li < 2))
    return x3


# speedup vs baseline: 16.1507x; 1.3102x over previous
import functools

import jax
import jax.numpy as jnp
from jax import lax
from jax.experimental import pallas as pl
from jax.experimental.pallas import tpu as pltpu
from jax.experimental.pallas import tpu_sc as plsc

N = 100000
D = 32
DH = D // 2
E = 1600000
EPS = 1e-5

DUMMY = N
ACC_ROWS = 100096
TROWS = ACC_ROWS // 16

DEG_ROWS = 100352
DTROWS = DEG_ROWS // 16

LANES = 128
R_TILE = 784
R_TOT = R_TILE * 16
CHUNK = 4
NCHUNK = R_TILE // CHUNK
NPAIR = NCHUNK // 2
E_PAD = R_TOT * LANES

_mesh = plsc.VectorSubcoreMesh(core_axis_name="c", subcore_axis_name="s")


def _zero16():
    return jnp.zeros((16,), jnp.float32)


@functools.partial(
    pl.kernel,
    out_type=[
        jax.ShapeDtypeStruct((2, ACC_ROWS, DH), jnp.float32),
        jax.ShapeDtypeStruct((R_TOT, LANES), jnp.int32),
        jax.ShapeDtypeStruct((2, DEG_ROWS), jnp.float32),
    ],
    mesh=_mesh,
    compiler_params=pltpu.CompilerParams(use_tc_tiling_on_sc=False),
    scratch_types=[
        pltpu.VMEM_SHARED((ACC_ROWS, DH), jnp.float32),
        pltpu.VMEM_SHARED((DEG_ROWS,), jnp.float32),
        pltpu.VMEM((184, DH), jnp.float32),
        pltpu.VMEM((1568,), jnp.float32),
        pltpu.VMEM((2 * CHUNK, LANES), jnp.int32),
        pltpu.VMEM((2 * CHUNK, LANES), jnp.int32),
        pltpu.VMEM((CHUNK, LANES), jnp.int32),
        pltpu.VMEM((2 * CHUNK * LANES, DH), jnp.float32),
        pltpu.VMEM((LANES,), jnp.float32),
        pltpu.SemaphoreType.DMA,
        pltpu.SemaphoreType.DMA,
    ],
)
def _sc_agg_first(x3_hbm, row2d, col2d, out_hbm, colloc_out, deg_out,
                  acc, deg_sp, zbuf, zdeg, gi, si, cbuf, rows, ones_v,
                  sem0, sem1):
    ci = lax.axis_index("c")
    s = lax.axis_index("s")
    xv = x3_hbm.at[ci]
    sems = (sem0, sem1)

    @pl.loop(0, 184)
    def _zl(r):
        zbuf[r, pl.ds(0, 16)] = _zero16()

    @pl.loop(0, 98)
    def _zd(t):
        off = pl.multiple_of(t * 16, 16)
        zdeg[pl.ds(off, 16)] = _zero16()

    for v in range(LANES // 16):
        ones_v[pl.ds(v * 16, 16)] = jnp.full((16,), 1.0, jnp.float32)

    for q in range(TROWS // 184):
        pltpu.sync_copy(zbuf, acc.at[pl.ds(s * TROWS + q * 184, 184)])
    for q in range(DTROWS // 1568):
        pltpu.sync_copy(zdeg, deg_sp.at[pl.ds(s * DTROWS + q * 1568, 1568)])
    plsc.subcore_barrier()

    def _fire(b, k):
        rb = s * R_TILE + k * CHUNK
        pltpu.sync_copy(row2d.at[pl.ds(rb, CHUNK)],
                        gi.at[pl.ds(b * CHUNK, CHUNK)])
        pltpu.sync_copy(col2d.at[pl.ds(rb, CHUNK)], cbuf)
        for j in range(CHUNK):
            for v in range(LANES // 16):
                off = pl.multiple_of(v * 16, 16)
                rv = gi[b * CHUNK + j, pl.ds(off, 16)]
                cv = cbuf[j, pl.ds(off, 16)]
                si[b * CHUNK + j, pl.ds(off, 16)] = (
                    jnp.where(rv != cv, cv, DUMMY))
        pltpu.sync_copy(si.at[pl.ds(b * CHUNK, CHUNK)],
                        colloc_out.at[pl.ds(rb, CHUNK)])
        for j in range(CHUNK):
            pltpu.async_copy(
                xv.at[gi.at[b * CHUNK + j]],
                rows.at[pl.ds((b * CHUNK + j) * LANES, LANES)], sems[b])

    def _drain_and_scatter(b, k):
        for j in range(CHUNK):
            pltpu.make_async_copy(
                xv.at[gi.at[b * CHUNK + j]],
                rows.at[pl.ds((b * CHUNK + j) * LANES, LANES)],
                sems[b]).wait()
        for j in range(CHUNK):
            pltpu.sync_copy(rows.at[pl.ds((b * CHUNK + j) * LANES, LANES)],
                            acc.at[si.at[b * CHUNK + j]], add=True)
        half = k < NCHUNK // 2

        @pl.when((half & (ci == 0)) | (jnp.logical_not(half) & (ci == 1)))
        def _deg():
            for j in range(CHUNK):
                pltpu.sync_copy(ones_v, deg_sp.at[si.at[b * CHUNK + j]],
                                add=True)

    _fire(0, 0)

    @pl.loop(0, NPAIR)
    def _pair(kk):
        _fire(1, 2 * kk + 1)
        _drain_and_scatter(0, 2 * kk)

        @pl.when(kk < NPAIR - 1)
        def _next():
            _fire(0, 2 * kk + 2)

        _drain_and_scatter(1, 2 * kk + 1)

    plsc.subcore_barrier()
    pltpu.sync_copy(acc.at[pl.ds(s * TROWS, TROWS)],
                    out_hbm.at[ci, pl.ds(s * TROWS, TROWS)])
    pltpu.sync_copy(deg_sp.at[pl.ds(s * DTROWS, DTROWS)],
                    deg_out.at[ci, pl.ds(s * DTROWS, DTROWS)])


@functools.partial(
    pl.kernel,
    out_type=jax.ShapeDtypeStruct((2, ACC_ROWS, DH), jnp.float32),
    mesh=_mesh,
    compiler_params=pltpu.CompilerParams(use_tc_tiling_on_sc=False),
    scratch_types=[
        pltpu.VMEM_SHARED((ACC_ROWS, DH), jnp.float32),
        pltpu.VMEM((184, DH), jnp.float32),
        pltpu.VMEM((2 * CHUNK, LANES), jnp.int32),
        pltpu.VMEM((2 * CHUNK, LANES), jnp.int32),
        pltpu.VMEM((2 * CHUNK * LANES, DH), jnp.float32),
        pltpu.SemaphoreType.DMA,
        pltpu.SemaphoreType.DMA,
    ],
)
def _sc_agg(x3_hbm, row2d, colloc, out_hbm, acc, zbuf, gi, si, rows,
            sem0, sem1):
    ci = lax.axis_index("c")
    s = lax.axis_index("s")
    xv = x3_hbm.at[ci]
    sems = (sem0, sem1)

    @pl.loop(0, 184)
    def _zl(r):
        zbuf[r, pl.ds(0, 16)] = _zero16()

    for q in range(TROWS // 184):
        pltpu.sync_copy(zbuf, acc.at[pl.ds(s * TROWS + q * 184, 184)])
    plsc.subcore_barrier()

    def _fire(b, k):
        rb = s * R_TILE + k * CHUNK
        pltpu.sync_copy(row2d.at[pl.ds(rb, CHUNK)],
                        gi.at[pl.ds(b * CHUNK, CHUNK)])
        pltpu.sync_copy(colloc.at[pl.ds(rb, CHUNK)],
                        si.at[pl.ds(b * CHUNK, CHUNK)])
        for j in range(CHUNK):
            pltpu.async_copy(
                xv.at[gi.at[b * CHUNK + j]],
                rows.at[pl.ds((b * CHUNK + j) * LANES, LANES)], sems[b])

    def _drain_and_scatter(b):
        for j in range(CHUNK):
            pltpu.make_async_copy(
                xv.at[gi.at[b * CHUNK + j]],
                rows.at[pl.ds((b * CHUNK + j) * LANES, LANES)],
                sems[b]).wait()
        for j in range(CHUNK):
            pltpu.sync_copy(rows.at[pl.ds((b * CHUNK + j) * LANES, LANES)],
                            acc.at[si.at[b * CHUNK + j]], add=True)

    _fire(0, 0)

    @pl.loop(0, NPAIR)
    def _pair(kk):
        _fire(1, 2 * kk + 1)
        _drain_and_scatter(0)

        @pl.when(kk < NPAIR - 1)
        def _next():
            _fire(0, 2 * kk + 2)

        _drain_and_scatter(1)

    plsc.subcore_barrier()
    pltpu.sync_copy(acc.at[pl.ds(s * TROWS, TROWS)],
                    out_hbm.at[ci, pl.ds(s * TROWS, TROWS)])


_BN = 2000
_GRID = N // _BN


def _combine_body(act, x3_ref, r3_ref, cnt_ref, wo_ref, b_ref, wr_ref,
                  y_ref, st_ref, acc_ref):
    i = pl.program_id(0)
    xb = jnp.concatenate([x3_ref[0], x3_ref[1]], axis=1)
    rawb = jnp.concatenate([r3_ref[0], r3_ref[1]], axis=1)
    inv = 1.0 / (cnt_ref[...] + 1.0)
    t = (rawb + xb) * inv
    h = (jnp.dot(t, wo_ref[...], preferred_element_type=jnp.float32)
         + b_ref[...]
         + jnp.dot(xb, wr_ref[...], preferred_element_type=jnp.float32))
    if act:
        h = jnp.where(h >= 0, h, 0.01 * h)
    y = h + xb
    y_ref[...] = y

    @pl.when(i == 0)
    def _init():
        acc_ref[...] = jnp.zeros_like(acc_ref)

    acc_ref[0:1, :] = acc_ref[0:1, :] + jnp.sum(y, axis=0, keepdims=True)
    acc_ref[1:2, :] = acc_ref[1:2, :] + jnp.sum(y * y, axis=0, keepdims=True)

    @pl.when(i == _GRID - 1)
    def _fin():
        st_ref[...] = acc_ref[...]


def _tc_combine(x3, raw3, cnt, wo, b, wr, act):
    return pl.pallas_call(
        functools.partial(_combine_body, act),
        grid=(_GRID,),
        in_specs=[
            pl.BlockSpec((2, _BN, DH), lambda i: (0, i, 0)),
            pl.BlockSpec((2, _BN, DH), lambda i: (0, i, 0)),
            pl.BlockSpec((_BN, 1), lambda i: (i, 0)),
            pl.BlockSpec((D, D), lambda i: (0, 0)),
            pl.BlockSpec((1, D), lambda i: (0, 0)),
            pl.BlockSpec((D, D), lambda i: (0, 0)),
        ],
        out_specs=[
            pl.BlockSpec((_BN, D), lambda i: (i, 0)),
            pl.BlockSpec((2, D), lambda i: (0, 0)),
        ],
        out_shape=[
            jax.ShapeDtypeStruct((N, D), jnp.float32),
            jax.ShapeDtypeStruct((2, D), jnp.float32),
        ],
        scratch_shapes=[pltpu.VMEM((2, D), jnp.float32)],
    )(x3, raw3, cnt, wo, b, wr)


def _norm_body_split(y_ref, st_ref, g_ref, be_ref, o3_ref):
    st = st_ref[...]
    mean = st[0:1, :] / N
    var = st[1:2, :] / N - mean * mean
    scale = g_ref[...] / jnp.sqrt(var + EPS)
    z = (y_ref[...] - mean) * scale + be_ref[...]
    o3_ref[0] = z[:, :DH]
    o3_ref[1] = z[:, DH:]


def _norm_body_dense(y_ref, st_ref, g_ref, be_ref, o_ref):
    st = st_ref[...]
    mean = st[0:1, :] / N
    var = st[1:2, :] / N - mean * mean
    scale = g_ref[...] / jnp.sqrt(var + EPS)
    o_ref[...] = (y_ref[...] - mean) * scale + be_ref[...]


def _tc_norm(y, st, g, be, split):
    if split:
        body = _norm_body_split
        out_spec = pl.BlockSpec((2, _BN, DH), lambda i: (0, i, 0))
        out_shape = jax.ShapeDtypeStruct((2, N, DH), jnp.float32)
    else:
        body = _norm_body_dense
        out_spec = pl.BlockSpec((_BN, D), lambda i: (i, 0))
        out_shape = jax.ShapeDtypeStruct((N, D), jnp.float32)
    return pl.pallas_call(
        body,
        grid=(_GRID,),
        in_specs=[
            pl.BlockSpec((_BN, D), lambda i: (i, 0)),
            pl.BlockSpec((2, D), lambda i: (0, 0)),
            pl.BlockSpec((1, D), lambda i: (0, 0)),
            pl.BlockSpec((1, D), lambda i: (0, 0)),
        ],
        out_specs=out_spec,
        out_shape=out_shape,
    )(y, st, g, be)


def kernel(patch_embs, edge_index, edge_attr,
           W1_out, b1_out, W1_root, g1, be1,
           W2_out, b2_out, W2_root, g2, be2,
           W3_out, b3_out, W3_root, g3, be3):
    row = edge_index[0].astype(jnp.int32)
    col = edge_index[1].astype(jnp.int32)
    pad = E_PAD - E
    row_p = jnp.concatenate([row, jnp.zeros((pad,), jnp.int32)])
    col_p = jnp.concatenate([col, jnp.full((pad,), N, jnp.int32)])
    row2d = row_p.reshape(R_TOT, LANES)
    col2d = col_p.reshape(R_TOT, LANES)

    x3 = jnp.stack([patch_embs[:, :DH], patch_embs[:, DH:]])
    raw3, colloc, degp = _sc_agg_first(x3, row2d, col2d)
    cnt = (degp[0, :N] + degp[1, :N]).reshape(N, 1)

    layers = [
        (W1_out, b1_out, W1_root, g1, be1, True),
        (W2_out, b2_out, W2_root, g2, be2, True),
        (W3_out, b3_out, W3_root, g3, be3, False),
    ]
    for li, (wo, b, wr, g, be, act) in enumerate(layers):
        if li > 0:
            raw3 = _sc_agg(x3, row2d, colloc)
        y, st = _tc_combine(x3, raw3, cnt, wo, b.reshape(1, D), wr, act)
        x3 = _tc_norm(y, st, g.reshape(1, D), be.reshape(1, D),
                      split=(li < 2))
    return x3


# speedup vs baseline: 16.7038x; 1.0342x over previous
import functools

import jax
import jax.numpy as jnp
from jax import lax
from jax.experimental import pallas as pl
from jax.experimental.pallas import tpu as pltpu
from jax.experimental.pallas import tpu_sc as plsc

N = 100000
D = 32
DH = D // 2
E = 1600000
EPS = 1e-5

DUMMY = N
ACC_ROWS = 100096
TROWS = ACC_ROWS // 16

DEG_ROWS = 100352
DTROWS = DEG_ROWS // 16

LANES = 128
R_TILE = 784
R_TOT = R_TILE * 16
CHUNK = 4
NCHUNK = R_TILE // CHUNK
NPAIR = NCHUNK // 2
E_PAD = R_TOT * LANES

_mesh = plsc.VectorSubcoreMesh(core_axis_name="c", subcore_axis_name="s")


def _zero16():
    return jnp.zeros((16,), jnp.float32)


@functools.partial(
    pl.kernel,
    out_type=[
        jax.ShapeDtypeStruct((2, ACC_ROWS, DH), jnp.float32),
        jax.ShapeDtypeStruct((R_TOT, LANES), jnp.int32),
        jax.ShapeDtypeStruct((2, DEG_ROWS), jnp.float32),
    ],
    mesh=_mesh,
    compiler_params=pltpu.CompilerParams(use_tc_tiling_on_sc=False),
    scratch_types=[
        pltpu.VMEM_SHARED((ACC_ROWS, DH), jnp.float32),
        pltpu.VMEM_SHARED((DEG_ROWS,), jnp.float32),
        pltpu.VMEM((184, DH), jnp.float32),
        pltpu.VMEM((1568,), jnp.float32),
        pltpu.VMEM((2 * CHUNK, LANES), jnp.int32),
        pltpu.VMEM((2 * CHUNK, LANES), jnp.int32),
        pltpu.VMEM((CHUNK, LANES), jnp.int32),
        pltpu.VMEM((2 * CHUNK * LANES, DH), jnp.float32),
        pltpu.VMEM((LANES,), jnp.float32),
        pltpu.SemaphoreType.DMA,
        pltpu.SemaphoreType.DMA,
    ],
)
def _sc_agg_first(x3_hbm, row2d, col2d, out_hbm, colloc_out, deg_out,
                  acc, deg_sp, zbuf, zdeg, gi, si, cbuf, rows, ones_v,
                  sem0, sem1):
    ci = lax.axis_index("c")
    s = lax.axis_index("s")
    xv = x3_hbm.at[ci]
    sems = (sem0, sem1)

    @pl.loop(0, 184)
    def _zl(r):
        zbuf[r, pl.ds(0, 16)] = _zero16()

    @pl.loop(0, 98)
    def _zd(t):
        off = pl.multiple_of(t * 16, 16)
        zdeg[pl.ds(off, 16)] = _zero16()

    for v in range(LANES // 16):
        ones_v[pl.ds(v * 16, 16)] = jnp.full((16,), 1.0, jnp.float32)

    for q in range(TROWS // 184):
        pltpu.sync_copy(zbuf, acc.at[pl.ds(s * TROWS + q * 184, 184)])
    for q in range(DTROWS // 1568):
        pltpu.sync_copy(zdeg, deg_sp.at[pl.ds(s * DTROWS + q * 1568, 1568)])
    plsc.subcore_barrier()

    def _fire(b, k):
        rb = s * R_TILE + k * CHUNK
        pltpu.sync_copy(row2d.at[pl.ds(rb, CHUNK)],
                        gi.at[pl.ds(b * CHUNK, CHUNK)])
        pltpu.sync_copy(col2d.at[pl.ds(rb, CHUNK)], cbuf)
        for j in range(CHUNK):
            for v in range(LANES // 16):
                off = pl.multiple_of(v * 16, 16)
                rv = gi[b * CHUNK + j, pl.ds(off, 16)]
                cv = cbuf[j, pl.ds(off, 16)]
                si[b * CHUNK + j, pl.ds(off, 16)] = (
                    jnp.where(rv != cv, cv, DUMMY))
        pltpu.sync_copy(si.at[pl.ds(b * CHUNK, CHUNK)],
                        colloc_out.at[pl.ds(rb, CHUNK)])
        for j in range(CHUNK):
            pltpu.async_copy(
                xv.at[gi.at[b * CHUNK + j]],
                rows.at[pl.ds((b * CHUNK + j) * LANES, LANES)], sems[b])

    def _drain_and_scatter(b, k):
        for j in range(CHUNK):
            pltpu.make_async_copy(
                xv.at[gi.at[b * CHUNK + j]],
                rows.at[pl.ds((b * CHUNK + j) * LANES, LANES)],
                sems[b]).wait()
        for j in range(CHUNK):
            pltpu.sync_copy(rows.at[pl.ds((b * CHUNK + j) * LANES, LANES)],
                            acc.at[si.at[b * CHUNK + j]], add=True)
        half = k < NCHUNK // 2

        @pl.when((half & (ci == 0)) | (jnp.logical_not(half) & (ci == 1)))
        def _deg():
            for j in range(CHUNK):
                pltpu.sync_copy(ones_v, deg_sp.at[si.at[b * CHUNK + j]],
                                add=True)

    _fire(0, 0)

    @pl.loop(0, NPAIR)
    def _pair(kk):
        _fire(1, 2 * kk + 1)
        _drain_and_scatter(0, 2 * kk)

        @pl.when(kk < NPAIR - 1)
        def _next():
            _fire(0, 2 * kk + 2)

        _drain_and_scatter(1, 2 * kk + 1)

    plsc.subcore_barrier()
    pltpu.sync_copy(acc.at[pl.ds(s * TROWS, TROWS)],
                    out_hbm.at[ci, pl.ds(s * TROWS, TROWS)])
    pltpu.sync_copy(deg_sp.at[pl.ds(s * DTROWS, DTROWS)],
                    deg_out.at[ci, pl.ds(s * DTROWS, DTROWS)])


@functools.partial(
    pl.kernel,
    out_type=jax.ShapeDtypeStruct((2, ACC_ROWS, DH), jnp.float32),
    mesh=_mesh,
    compiler_params=pltpu.CompilerParams(use_tc_tiling_on_sc=False),
    scratch_types=[
        pltpu.VMEM_SHARED((ACC_ROWS, DH), jnp.float32),
        pltpu.VMEM((184, DH), jnp.float32),
        pltpu.VMEM((2 * CHUNK, LANES), jnp.int32),
        pltpu.VMEM((2 * CHUNK, LANES), jnp.int32),
        pltpu.VMEM((2 * CHUNK * LANES, DH), jnp.float32),
        pltpu.SemaphoreType.DMA,
        pltpu.SemaphoreType.DMA,
    ],
)
def _sc_agg(x3_hbm, row2d, colloc, out_hbm, acc, zbuf, gi, si, rows,
            sem0, sem1):
    ci = lax.axis_index("c")
    s = lax.axis_index("s")
    xv = x3_hbm.at[ci]
    sems = (sem0, sem1)

    @pl.loop(0, 184)
    def _zl(r):
        zbuf[r, pl.ds(0, 16)] = _zero16()

    for q in range(TROWS // 184):
        pltpu.sync_copy(zbuf, acc.at[pl.ds(s * TROWS + q * 184, 184)])
    plsc.subcore_barrier()

    def _fire(b, k):
        rb = s * R_TILE + k * CHUNK
        pltpu.sync_copy(row2d.at[pl.ds(rb, CHUNK)],
                        gi.at[pl.ds(b * CHUNK, CHUNK)])
        pltpu.sync_copy(colloc.at[pl.ds(rb, CHUNK)],
                        si.at[pl.ds(b * CHUNK, CHUNK)])
        for j in range(CHUNK):
            pltpu.async_copy(
                xv.at[gi.at[b * CHUNK + j]],
                rows.at[pl.ds((b * CHUNK + j) * LANES, LANES)], sems[b])

    def _drain_and_scatter(b):
        for j in range(CHUNK):
            pltpu.make_async_copy(
                xv.at[gi.at[b * CHUNK + j]],
                rows.at[pl.ds((b * CHUNK + j) * LANES, LANES)],
                sems[b]).wait()
        for j in range(CHUNK):
            pltpu.sync_copy(rows.at[pl.ds((b * CHUNK + j) * LANES, LANES)],
                            acc.at[si.at[b * CHUNK + j]], add=True)

    _fire(0, 0)

    @pl.loop(0, NPAIR)
    def _pair(kk):
        _fire(1, 2 * kk + 1)
        _drain_and_scatter(0)

        @pl.when(kk < NPAIR - 1)
        def _next():
            _fire(0, 2 * kk + 2)

        _drain_and_scatter(1)

    plsc.subcore_barrier()
    pltpu.sync_copy(acc.at[pl.ds(s * TROWS, TROWS)],
                    out_hbm.at[ci, pl.ds(s * TROWS, TROWS)])


_BN = 2000
_GRID = N // _BN


def _affine(st, g, be):
    mean = st[0:1, :] / N
    var = st[1:2, :] / N - mean * mean
    a = g / jnp.sqrt(var + EPS)
    return a, be - mean * a


def _combine_body(act, y3_ref, r3_ref, cnt_ref, stp_ref, gp_ref, bep_ref,
                  wo_ref, b_ref, wr_ref, o3_ref, st_ref, acc_ref):
    i = pl.program_id(0)
    yb = jnp.concatenate([y3_ref[0], y3_ref[1]], axis=1)
    rawb = jnp.concatenate([r3_ref[0], r3_ref[1]], axis=1)
    a, bo = _affine(stp_ref[...], gp_ref[...], bep_ref[...])
    xb = yb * a + bo
    inv = 1.0 / (cnt_ref[...] + 1.0)
    t = a * ((rawb + yb) * inv) + bo
    h = (jnp.dot(t, wo_ref[...], preferred_element_type=jnp.float32)
         + b_ref[...]
         + jnp.dot(xb, wr_ref[...], preferred_element_type=jnp.float32))
    if act:
        h = jnp.where(h >= 0, h, 0.01 * h)
    y = h + xb
    o3_ref[0] = y[:, :DH]
    o3_ref[1] = y[:, DH:]

    @pl.when(i == 0)
    def _init():
        acc_ref[...] = jnp.zeros_like(acc_ref)

    acc_ref[0:1, :] = acc_ref[0:1, :] + jnp.sum(y, axis=0, keepdims=True)
    acc_ref[1:2, :] = acc_ref[1:2, :] + jnp.sum(y * y, axis=0, keepdims=True)

    @pl.when(i == _GRID - 1)
    def _fin():
        st_ref[...] = acc_ref[...]


def _tc_combine(y3, raw3, cnt, stp, gp, bep, wo, b, wr, act):
    return pl.pallas_call(
        functools.partial(_combine_body, act),
        grid=(_GRID,),
        in_specs=[
            pl.BlockSpec((2, _BN, DH), lambda i: (0, i, 0)),
            pl.BlockSpec((2, _BN, DH), lambda i: (0, i, 0)),
            pl.BlockSpec((_BN, 1), lambda i: (i, 0)),
            pl.BlockSpec((2, D), lambda i: (0, 0)),
            pl.BlockSpec((1, D), lambda i: (0, 0)),
            pl.BlockSpec((1, D), lambda i: (0, 0)),
            pl.BlockSpec((D, D), lambda i: (0, 0)),
            pl.BlockSpec((1, D), lambda i: (0, 0)),
            pl.BlockSpec((D, D), lambda i: (0, 0)),
        ],
        out_specs=[
            pl.BlockSpec((2, _BN, DH), lambda i: (0, i, 0)),
            pl.BlockSpec((2, D), lambda i: (0, 0)),
        ],
        out_shape=[
            jax.ShapeDtypeStruct((2, N, DH), jnp.float32),
            jax.ShapeDtypeStruct((2, D), jnp.float32),
        ],
        scratch_shapes=[pltpu.VMEM((2, D), jnp.float32)],
    )(y3, raw3, cnt, stp, gp, bep, wo, b, wr)


def _norm_body(y3_ref, st_ref, g_ref, be_ref, o_ref):
    yb = jnp.concatenate([y3_ref[0], y3_ref[1]], axis=1)
    a, bo = _affine(st_ref[...], g_ref[...], be_ref[...])
    o_ref[...] = yb * a + bo


def _tc_norm(y3, st, g, be):
    return pl.pallas_call(
        _norm_body,
        grid=(_GRID,),
        in_specs=[
            pl.BlockSpec((2, _BN, DH), lambda i: (0, i, 0)),
            pl.BlockSpec((2, D), lambda i: (0, 0)),
            pl.BlockSpec((1, D), lambda i: (0, 0)),
            pl.BlockSpec((1, D), lambda i: (0, 0)),
        ],
        out_specs=pl.BlockSpec((_BN, D), lambda i: (i, 0)),
        out_shape=jax.ShapeDtypeStruct((N, D), jnp.float32),
    )(y3, st, g, be)


def kernel(patch_embs, edge_index, edge_attr,
           W1_out, b1_out, W1_root, g1, be1,
           W2_out, b2_out, W2_root, g2, be2,
           W3_out, b3_out, W3_root, g3, be3):
    row = edge_index[0].astype(jnp.int32)
    col = edge_index[1].astype(jnp.int32)
    pad = E_PAD - E
    row_p = jnp.concatenate([row, jnp.zeros((pad,), jnp.int32)])
    col_p = jnp.concatenate([col, jnp.full((pad,), N, jnp.int32)])
    row2d = row_p.reshape(R_TOT, LANES)
    col2d = col_p.reshape(R_TOT, LANES)

    y3 = jnp.stack([patch_embs[:, :DH], patch_embs[:, DH:]])
    raw3, colloc, degp = _sc_agg_first(y3, row2d, col2d)
    cnt = (degp[0, :N] + degp[1, :N]).reshape(N, 1)

    st = jnp.stack([jnp.zeros((D,), jnp.float32),
                    jnp.full((D,), N * (1.0 - EPS), jnp.float32)])
    gp = jnp.ones((1, D), jnp.float32)
    bep = jnp.zeros((1, D), jnp.float32)

    layers = [
        (W1_out, b1_out, W1_root, True),
        (W2_out, b2_out, W2_root, True),
        (W3_out, b3_out, W3_root, False),
    ]
    for li, (wo, b, wr, act) in enumerate(layers):
        if li > 0:
            raw3 = _sc_agg(y3, row2d, colloc)
        y3, st = _tc_combine(y3, raw3, cnt, st, gp, bep,
                             wo, b.reshape(1, D), wr, act)
        gp, bep = [(g1, be1), (g2, be2), (g3, be3)][li]
        gp, bep = gp.reshape(1, D), bep.reshape(1, D)
    return _tc_norm(y3, st, gp, bep)
